# baseline (speedup 1.0000x reference)
import functools

import jax
import jax.numpy as jnp
from jax import lax
from jax.experimental import pallas as pl
from jax.experimental.pallas import tpu as pltpu
from jax.experimental.pallas import tpu_sc as plsc

K_NN = 32
TR = 256
CT = 512
EPS = 1e-5


def _topk_body(xr_ref, xa_ref, idx_ref):
    xr = xr_ref[0]
    xa = xa_ref[0]
    N = xa.shape[0]
    inner = lax.dot_general(xa, xr, (((1,), (1,)), ((), ())),
                            preferred_element_type=jnp.float32)
    sqr = jnp.sum(xr * xr, axis=1)
    sqa = jnp.sum(xa * xa, axis=1)
    dist = sqa[:, None] - 2.0 * inner + sqr[None, :]
    iota = lax.broadcasted_iota(jnp.int32, (N, TR), 0)
    iota_k = lax.broadcasted_iota(jnp.int32, (K_NN, TR), 0)

    def body(k, carry):
        dist, acc = carry
        m = jnp.min(dist, axis=0)
        eq = dist == m[None, :]
        sel = jnp.min(jnp.where(eq, iota, N), axis=0)
        acc = jnp.where(iota_k == k, sel[None, :], acc)
        dist = jnp.where(iota == sel[None, :], jnp.inf, dist)
        return dist, acc

    _, acc = lax.fori_loop(
        0, K_NN, body, (dist, jnp.zeros((K_NN, TR), jnp.int32)))
    idx_ref[0] = acc


def _topk_idx(xt4):
    B, N, _ = xt4.shape
    kt = pl.pallas_call(
        _topk_body,
        out_shape=jax.ShapeDtypeStruct((B, K_NN, N), jnp.int32),
        grid=(B, N // TR),
        in_specs=[
            pl.BlockSpec((1, TR, 4), lambda b, t: (b, t, 0)),
            pl.BlockSpec((1, N, 4), lambda b, t: (b, 0, 0)),
        ],
        out_specs=pl.BlockSpec((1, K_NN, TR), lambda b, t: (b, 0, t)),
    )(xt4, xt4)
    return jnp.transpose(kt, (0, 2, 1))


def _sc_gather(x, idx_flat):
    B, _, N = x.shape
    NW = 32
    pts_w = (B * N) // NW
    idx_w = pts_w * K_NN
    mesh = plsc.VectorSubcoreMesh(core_axis_name="c", subcore_axis_name="s")

    @functools.partial(
        pl.kernel, mesh=mesh,
        out_type=jax.ShapeDtypeStruct((B * N * K_NN * 4,), jnp.float32),
        compiler_params=pltpu.CompilerParams(needs_layout_passes=False),
        scratch_types=[
            pltpu.VMEM((idx_w,), jnp.int32),
            pltpu.VMEM((N,), jnp.float32),
            pltpu.VMEM((N,), jnp.float32),
            pltpu.VMEM((N,), jnp.float32),
            pltpu.VMEM((idx_w * 4,), jnp.float32),
        ],
    )
    def gk(x_hbm, idx_hbm, out_hbm, idx_v, tx, ty, tz, out_v):
        wid = lax.axis_index("s") * 2 + lax.axis_index("c")
        b = (wid * pts_w) // N
        pltpu.sync_copy(x_hbm.at[pl.ds((b * 3 + 0) * N, N)], tx)
        pltpu.sync_copy(x_hbm.at[pl.ds((b * 3 + 1) * N, N)], ty)
        pltpu.sync_copy(x_hbm.at[pl.ds((b * 3 + 2) * N, N)], tz)
        pltpu.sync_copy(idx_hbm.at[pl.ds(wid * idx_w, idx_w)], idx_v)
        lane4 = lax.broadcasted_iota(jnp.int32, (16,), 0) * 4
        zero16 = jnp.zeros((16,), jnp.float32)

        def body(i, _):
            iv = idx_v[pl.ds(i * 16, 16)]
            gx = plsc.load_gather(tx, [iv])
            gy = plsc.load_gather(ty, [iv])
            gz = plsc.load_gather(tz, [iv])
            pos = lane4 + i * 64
            plsc.store_scatter(out_v, [pos], gx)
            plsc.store_scatter(out_v, [pos + 1], gy)
            plsc.store_scatter(out_v, [pos + 2], gz)
            plsc.store_scatter(out_v, [pos + 3], zero16)
            return 0

        lax.fori_loop(0, idx_w // 16, body, 0)
        pltpu.sync_copy(out_v, out_hbm.at[pl.ds(wid * idx_w * 4, idx_w * 4)])

    return gk(x.reshape(-1), idx_flat)


def _edge_h(xt_ref, nb_ref):
    c = xt_ref[0]
    nbt = nb_ref[0]
    cb = jnp.broadcast_to(c[:, None, :], (CT, K_NN, 4)).reshape(CT * K_NN, 4)
    return jnp.concatenate([cb, nbt - cb], axis=1)


def _c1_body(xt_ref, nb_ref, sh_ref, shh_ref):
    @pl.when((pl.program_id(0) == 0) & (pl.program_id(1) == 0))
    def _():
        sh_ref[...] = jnp.zeros_like(sh_ref)
        shh_ref[...] = jnp.zeros_like(shh_ref)
    h = _edge_h(xt_ref, nb_ref)
    sh_ref[...] += jnp.sum(h, axis=0)[None, :]
    shh_ref[...] += lax.dot_general(h, h, (((0,), (0,)), ((), ())),
                                    preferred_element_type=jnp.float32)


def _c2_body(xt_ref, nb_ref, w1_ref, c1_ref, sz_ref, szz_ref):
    @pl.when((pl.program_id(0) == 0) & (pl.program_id(1) == 0))
    def _():
        sz_ref[...] = jnp.zeros_like(sz_ref)
        szz_ref[...] = jnp.zeros_like(szz_ref)
    h = _edge_h(xt_ref, nb_ref)
    y = lax.dot_general(h, w1_ref[...], (((1,), (0,)), ((), ())),
                        preferred_element_type=jnp.float32) + c1_ref[...]
    z = jnp.where(y > 0, y, 0.2 * y)
    sz_ref[...] += jnp.sum(z, axis=0)[None, :]
    szz_ref[...] += lax.dot_general(z, z, (((0,), (0,)), ((), ())),
                                    preferred_element_type=jnp.float32)


def _c3_body(xt_ref, nb_ref, w1_ref, c1_ref, w2_ref, c2_ref, o_ref):
    h = _edge_h(xt_ref, nb_ref)
    y = lax.dot_general(h, w1_ref[...], (((1,), (0,)), ((), ())),
                        preferred_element_type=jnp.float32) + c1_ref[...]
    z = jnp.where(y > 0, y, 0.2 * y)
    y2 = lax.dot_general(z, w2_ref[...], (((1,), (0,)), ((), ())),
                         preferred_element_type=jnp.float32) + c2_ref[...]
    z2 = jnp.where(y2 > 0, y2, 0.2 * y2)
    o_ref[0] = jnp.max(z2.reshape(CT, K_NN, 64), axis=1)


def _conv_stack(xt4, nb4, W1, g1, b1, W2, g2, b2):
    B, N, _ = xt4.shape
    NT = N // CT
    cnt = B * N * K_NN
    xt_spec = pl.BlockSpec((1, CT, 4), lambda b, t: (b, t, 0))
    nb_spec = pl.BlockSpec((1, CT * K_NN, 4), lambda b, t: (b, t, 0))
    full = lambda shape: pl.BlockSpec(shape, lambda b, t: tuple(0 for _ in shape))

    sh, shh = pl.pallas_call(
        _c1_body,
        out_shape=(jax.ShapeDtypeStruct((1, 8), jnp.float32),
                   jax.ShapeDtypeStruct((8, 8), jnp.float32)),
        grid=(B, NT),
        in_specs=[xt_spec, nb_spec],
        out_specs=(full((1, 8)), full((8, 8))),
    )(xt4, nb4)

    mh = sh[0] / cnt
    M2 = shh / cnt
    z31 = jnp.zeros((64, 1), jnp.float32)
    W1p = jnp.concatenate([W1[:, :3], z31, W1[:, 3:], z31], axis=1)
    mean1 = W1p @ mh
    ey2 = jnp.sum((W1p @ M2) * W1p, axis=1)
    var1 = ey2 - mean1 * mean1
    a1 = g1 / jnp.sqrt(var1 + EPS)
    c1v = b1 - mean1 * a1
    W1effT = (a1[:, None] * W1p).T

    sz, szz = pl.pallas_call(
        _c2_body,
        out_shape=(jax.ShapeDtypeStruct((1, 64), jnp.float32),
                   jax.ShapeDtypeStruct((64, 64), jnp.float32)),
        grid=(B, NT),
        in_specs=[xt_spec, nb_spec, full((8, 64)), full((1, 64))],
        out_specs=(full((1, 64)), full((64, 64))),
    )(xt4, nb4, W1effT, c1v[None])

    mz = sz[0] / cnt
    Mzz = szz / cnt
    mean2 = W2 @ mz
    ey2b = jnp.sum((W2 @ Mzz) * W2, axis=1)
    var2 = ey2b - mean2 * mean2
    a2 = g2 / jnp.sqrt(var2 + EPS)
    c2v = b2 - mean2 * a2
    W2effT = (a2[:, None] * W2).T

    out = pl.pallas_call(
        _c3_body,
        out_shape=jax.ShapeDtypeStruct((B, N, 64), jnp.float32),
        grid=(B, NT),
        in_specs=[xt_spec, nb_spec, full((8, 64)), full((1, 64)),
                  full((64, 64)), full((1, 64))],
        out_specs=pl.BlockSpec((1, CT, 64), lambda b, t: (b, t, 0)),
    )(xt4, nb4, W1effT, c1v[None], W2effT, c2v[None])
    return out


def kernel(x, W1, g1, b1, W2, g2, b2):
    B, C, N = x.shape
    xt = jnp.transpose(x, (0, 2, 1))
    xt4 = jnp.pad(xt, ((0, 0), (0, 0), (0, 1)))
    idx = _topk_idx(xt4)
    nb_flat = _sc_gather(x, idx.reshape(-1))
    nb4 = nb_flat.reshape(B, N * K_NN, 4)
    out = _conv_stack(xt4, nb4, W1, g1, b1, W2, g2, b2)
    return jnp.transpose(out, (0, 2, 1))


# speedup vs baseline: 1.1150x; 1.1150x over previous
import functools

import jax
import jax.numpy as jnp
from jax import lax
from jax.experimental import pallas as pl
from jax.experimental.pallas import tpu as pltpu
from jax.experimental.pallas import tpu_sc as plsc

K_NN = 32
TR = 512
CT = 512
EPS = 1e-5


def _topk_body(xr_ref, xa_ref, idx_ref):
    xr = xr_ref[0]
    xa = xa_ref[0]
    N = xa.shape[0]
    inner = lax.dot_general(xa, xr, (((1,), (1,)), ((), ())),
                            preferred_element_type=jnp.float32)
    sqr = jnp.sum(xr * xr, axis=1)
    sqa = jnp.sum(xa * xa, axis=1)
    dist = sqa[:, None] - 2.0 * inner + sqr[None, :]
    iota = lax.broadcasted_iota(jnp.int32, (N, TR), 0)
    iota_k = lax.broadcasted_iota(jnp.int32, (K_NN, TR), 0)

    def body(k, carry):
        dist, acc = carry
        m = jnp.min(dist, axis=0)
        eq = dist == m[None, :]
        sel = jnp.min(jnp.where(eq, iota, N), axis=0)
        acc = jnp.where(iota_k == k, sel[None, :], acc)
        dist = jnp.where(iota == sel[None, :], jnp.inf, dist)
        return dist, acc

    _, acc = lax.fori_loop(
        0, K_NN, body, (dist, jnp.zeros((K_NN, TR), jnp.int32)))
    idx_ref[0] = acc


def _topk_idx(xt4):
    B, N, _ = xt4.shape
    kt = pl.pallas_call(
        _topk_body,
        out_shape=jax.ShapeDtypeStruct((B, K_NN, N), jnp.int32),
        grid=(B, N // TR),
        in_specs=[
            pl.BlockSpec((1, TR, 4), lambda b, t: (b, t, 0)),
            pl.BlockSpec((1, N, 4), lambda b, t: (b, 0, 0)),
        ],
        out_specs=pl.BlockSpec((1, K_NN, TR), lambda b, t: (b, 0, t)),
    )(xt4, xt4)
    return jnp.transpose(kt, (0, 2, 1))


def _sc_gather(x, idx_flat):
    B, _, N = x.shape
    NW = 32
    pts_w = (B * N) // NW
    idx_w = pts_w * K_NN
    mesh = plsc.VectorSubcoreMesh(core_axis_name="c", subcore_axis_name="s")

    @functools.partial(
        pl.kernel, mesh=mesh,
        out_type=jax.ShapeDtypeStruct((B * N * K_NN * 4,), jnp.float32),
        compiler_params=pltpu.CompilerParams(needs_layout_passes=False),
        scratch_types=[
            pltpu.VMEM((idx_w,), jnp.int32),
            pltpu.VMEM((N,), jnp.float32),
            pltpu.VMEM((N,), jnp.float32),
            pltpu.VMEM((N,), jnp.float32),
            pltpu.VMEM((idx_w * 4,), jnp.float32),
        ],
    )
    def gk(x_hbm, idx_hbm, out_hbm, idx_v, tx, ty, tz, out_v):
        wid = lax.axis_index("s") * 2 + lax.axis_index("c")
        b = (wid * pts_w) // N
        pltpu.sync_copy(x_hbm.at[pl.ds((b * 3 + 0) * N, N)], tx)
        pltpu.sync_copy(x_hbm.at[pl.ds((b * 3 + 1) * N, N)], ty)
        pltpu.sync_copy(x_hbm.at[pl.ds((b * 3 + 2) * N, N)], tz)
        pltpu.sync_copy(idx_hbm.at[pl.ds(wid * idx_w, idx_w)], idx_v)
        lane4 = lax.broadcasted_iota(jnp.int32, (16,), 0) * 4
        zero16 = jnp.zeros((16,), jnp.float32)

        def body(i, _):
            iv = idx_v[pl.ds(i * 16, 16)]
            gx = plsc.load_gather(tx, [iv])
            gy = plsc.load_gather(ty, [iv])
            gz = plsc.load_gather(tz, [iv])
            pos = lane4 + i * 64
            plsc.store_scatter(out_v, [pos], gx)
            plsc.store_scatter(out_v, [pos + 1], gy)
            plsc.store_scatter(out_v, [pos + 2], gz)
            plsc.store_scatter(out_v, [pos + 3], zero16)
            return 0

        lax.fori_loop(0, idx_w // 16, body, 0)
        pltpu.sync_copy(out_v, out_hbm.at[pl.ds(wid * idx_w * 4, idx_w * 4)])

    return gk(x.reshape(-1), idx_flat)


def _edge_h(xt_ref, nb_ref):
    c = xt_ref[0]
    nbt = nb_ref[0]
    cb = jnp.broadcast_to(c[:, None, :], (CT, K_NN, 4)).reshape(CT * K_NN, 4)
    return jnp.concatenate([cb, nbt - cb], axis=1)


def _c1_body(xt_ref, nb_ref, sh_ref, shh_ref):
    @pl.when((pl.program_id(0) == 0) & (pl.program_id(1) == 0))
    def _():
        sh_ref[...] = jnp.zeros_like(sh_ref)
        shh_ref[...] = jnp.zeros_like(shh_ref)
    h = _edge_h(xt_ref, nb_ref)
    sh_ref[...] += jnp.sum(h, axis=0)[None, :]
    shh_ref[...] += lax.dot_general(h, h, (((0,), (0,)), ((), ())),
                                    preferred_element_type=jnp.float32)


def _c2_body(xt_ref, nb_ref, w1_ref, c1_ref, sz_ref, szz_ref):
    @pl.when((pl.program_id(0) == 0) & (pl.program_id(1) == 0))
    def _():
        sz_ref[...] = jnp.zeros_like(sz_ref)
        szz_ref[...] = jnp.zeros_like(szz_ref)
    h = _edge_h(xt_ref, nb_ref)
    y = lax.dot_general(h, w1_ref[...], (((1,), (0,)), ((), ())),
                        preferred_element_type=jnp.float32) + c1_ref[...]
    z = jnp.where(y > 0, y, 0.2 * y)
    sz_ref[...] += jnp.sum(z, axis=0)[None, :]
    szz_ref[...] += lax.dot_general(z, z, (((0,), (0,)), ((), ())),
                                    preferred_element_type=jnp.float32)


def _c3_body(xt_ref, nb_ref, w1_ref, c1_ref, w2_ref, c2_ref, o_ref):
    h = _edge_h(xt_ref, nb_ref)
    y = lax.dot_general(h, w1_ref[...], (((1,), (0,)), ((), ())),
                        preferred_element_type=jnp.float32) + c1_ref[...]
    z = jnp.where(y > 0, y, 0.2 * y)
    y2 = lax.dot_general(z, w2_ref[...], (((1,), (0,)), ((), ())),
                         preferred_element_type=jnp.float32) + c2_ref[...]
    z2 = jnp.where(y2 > 0, y2, 0.2 * y2)
    o_ref[0] = jnp.max(z2.reshape(CT, K_NN, 64), axis=1)


def _conv_stack(xt4, nb4, W1, g1, b1, W2, g2, b2):
    B, N, _ = xt4.shape
    NT = N // CT
    cnt = B * N * K_NN
    xt_spec = pl.BlockSpec((1, CT, 4), lambda b, t: (b, t, 0))
    nb_spec = pl.BlockSpec((1, CT * K_NN, 4), lambda b, t: (b, t, 0))
    full = lambda shape: pl.BlockSpec(shape, lambda b, t: tuple(0 for _ in shape))

    sh, shh = pl.pallas_call(
        _c1_body,
        out_shape=(jax.ShapeDtypeStruct((1, 8), jnp.float32),
                   jax.ShapeDtypeStruct((8, 8), jnp.float32)),
        grid=(B, NT),
        in_specs=[xt_spec, nb_spec],
        out_specs=(full((1, 8)), full((8, 8))),
    )(xt4, nb4)

    mh = sh[0] / cnt
    M2 = shh / cnt
    z31 = jnp.zeros((64, 1), jnp.float32)
    W1p = jnp.concatenate([W1[:, :3], z31, W1[:, 3:], z31], axis=1)
    mean1 = W1p @ mh
    ey2 = jnp.sum((W1p @ M2) * W1p, axis=1)
    var1 = ey2 - mean1 * mean1
    a1 = g1 / jnp.sqrt(var1 + EPS)
    c1v = b1 - mean1 * a1
    W1effT = (a1[:, None] * W1p).T

    sz, szz = pl.pallas_call(
        _c2_body,
        out_shape=(jax.ShapeDtypeStruct((1, 64), jnp.float32),
                   jax.ShapeDtypeStruct((64, 64), jnp.float32)),
        grid=(B, NT),
        in_specs=[xt_spec, nb_spec, full((8, 64)), full((1, 64))],
        out_specs=(full((1, 64)), full((64, 64))),
    )(xt4, nb4, W1effT, c1v[None])

    mz = sz[0] / cnt
    Mzz = szz / cnt
    mean2 = W2 @ mz
    ey2b = jnp.sum((W2 @ Mzz) * W2, axis=1)
    var2 = ey2b - mean2 * mean2
    a2 = g2 / jnp.sqrt(var2 + EPS)
    c2v = b2 - mean2 * a2
    W2effT = (a2[:, None] * W2).T

    out = pl.pallas_call(
        _c3_body,
        out_shape=jax.ShapeDtypeStruct((B, N, 64), jnp.float32),
        grid=(B, NT),
        in_specs=[xt_spec, nb_spec, full((8, 64)), full((1, 64)),
                  full((64, 64)), full((1, 64))],
        out_specs=pl.BlockSpec((1, CT, 64), lambda b, t: (b, t, 0)),
    )(xt4, nb4, W1effT, c1v[None], W2effT, c2v[None])
    return out


def kernel(x, W1, g1, b1, W2, g2, b2):
    B, C, N = x.shape
    xt = jnp.transpose(x, (0, 2, 1))
    xt4 = jnp.pad(xt, ((0, 0), (0, 0), (0, 1)))
    idx = _topk_idx(xt4)
    nb_flat = _sc_gather(x, idx.reshape(-1))
    nb4 = nb_flat.reshape(B, N * K_NN, 4)
    out = _conv_stack(xt4, nb4, W1, g1, b1, W2, g2, b2)
    return jnp.transpose(out, (0, 2, 1))
